# baseline (speedup 1.0000x reference)
import jax
import jax.numpy as jnp
from jax.experimental import pallas as pl
from jax.experimental.pallas import tpu as pltpu


def _round_up(x, m):
    return ((x + m - 1) // m) * m


def _gcn_body(adj_ref, x_ref, w_ref, b_ref, o_ref, s_ref):
    s_ref[...] = jnp.dot(
        x_ref[...], w_ref[...], preferred_element_type=jnp.float32
    )
    o_ref[...] = (
        jnp.dot(adj_ref[...], s_ref[...], preferred_element_type=jnp.float32)
        + b_ref[...]
    )


def kernel(x, w, adj, b):
    N, F_in = x.shape
    F_out = w.shape[1]
    Nk = adj.shape[1]

    b2d = jnp.asarray(b, jnp.float32).reshape(1, F_out)

    tm = 256
    Nr = _round_up(N, tm)
    adj_p = jnp.pad(adj, ((0, Nr - N), (0, 0))) if Nr != N else adj
    x_p = jnp.pad(x, ((0, Nk - N), (0, 0))) if Nk != N else x

    out = pl.pallas_call(
        _gcn_body,
        out_shape=jax.ShapeDtypeStruct((Nr, F_out), jnp.float32),
        grid_spec=pltpu.PrefetchScalarGridSpec(
            num_scalar_prefetch=0,
            grid=(Nr // tm,),
            in_specs=[
                pl.BlockSpec((tm, Nk), lambda i: (i, 0)),
                pl.BlockSpec((Nk, F_in), lambda i: (0, 0)),
                pl.BlockSpec((F_in, F_out), lambda i: (0, 0)),
                pl.BlockSpec((1, F_out), lambda i: (0, 0)),
            ],
            out_specs=pl.BlockSpec((tm, F_out), lambda i: (i, 0)),
            scratch_shapes=[pltpu.VMEM((Nk, F_out), jnp.float32)],
        ),
        compiler_params=pltpu.CompilerParams(
            dimension_semantics=("parallel",),
            vmem_limit_bytes=60 << 20,
        ),
    )(adj_p, x_p, w, b2d)
    return out[:N] if Nr != N else out


# speedup vs baseline: 1.1418x; 1.1418x over previous
import jax
import jax.numpy as jnp
from jax.experimental import pallas as pl
from jax.experimental.pallas import tpu as pltpu


def _round_up(x, m):
    return ((x + m - 1) // m) * m


def _gcn_body(adj_ref, x_ref, w_ref, b_ref, o_ref, s_ref):
    s_ref[...] = jnp.dot(
        x_ref[...], w_ref[...], preferred_element_type=jnp.float32
    )
    o_ref[...] = (
        jnp.dot(adj_ref[...], s_ref[...], preferred_element_type=jnp.float32)
        + b_ref[...]
    )


def kernel(x, w, adj, b):
    N, F_in = x.shape
    F_out = w.shape[1]
    Nk = adj.shape[1]

    b2d = jnp.asarray(b, jnp.float32).reshape(1, F_out)

    tm = 512
    Nr = _round_up(N, tm)
    adj_p = jnp.pad(adj, ((0, Nr - N), (0, 0))) if Nr != N else adj
    x_p = jnp.pad(x, ((0, Nk - N), (0, 0))) if Nk != N else x

    out = pl.pallas_call(
        _gcn_body,
        out_shape=jax.ShapeDtypeStruct((Nr, F_out), jnp.float32),
        grid_spec=pltpu.PrefetchScalarGridSpec(
            num_scalar_prefetch=0,
            grid=(Nr // tm,),
            in_specs=[
                pl.BlockSpec((tm, Nk), lambda i: (i, 0)),
                pl.BlockSpec((Nk, F_in), lambda i: (0, 0)),
                pl.BlockSpec((F_in, F_out), lambda i: (0, 0)),
                pl.BlockSpec((1, F_out), lambda i: (0, 0)),
            ],
            out_specs=pl.BlockSpec((tm, F_out), lambda i: (i, 0)),
            scratch_shapes=[pltpu.VMEM((Nk, F_out), jnp.float32)],
        ),
        compiler_params=pltpu.CompilerParams(
            dimension_semantics=("parallel",),
            vmem_limit_bytes=60 << 20,
        ),
    )(adj_p, x_p, w, b2d)
    return out[:N] if Nr != N else out
